# initial kernel scaffold (unmeasured)
import jax
import jax.numpy as jnp
from jax import lax
from jax.experimental import pallas as pl
from jax.experimental.pallas import tpu as pltpu

N_DEV = 4
SQ = 2048
SKV = 2048
DM = 1024
HQ_TOTAL = 32
HQ_PER = 8
DH = 128
SCALE = 0.08838834764831843

QBLK = 256
N_QB = SQ // QBLK


def _gather_weights(wq, wo):

    def body(wq_ref, wo_ref, out_ref, send_sems, recv_sems):
        my = lax.axis_index("i")
        left = lax.rem(my + N_DEV - 1, N_DEV)
        right = lax.rem(my + 1, N_DEV)

        barrier = pltpu.get_barrier_semaphore()
        for nbr in (left, right):
            pl.semaphore_signal(
                barrier, inc=1, device_id=(nbr,),
                device_id_type=pl.DeviceIdType.MESH,
            )
        pl.semaphore_wait(barrier, 2)

        out_ref[pl.ds(my, 1), pl.ds(0, 1)] = wq_ref[...].reshape(1, 1, DM, DM)
        out_ref[pl.ds(my, 1), pl.ds(1, 1)] = wo_ref[...].reshape(1, 1, DM, DM)

        for h in range(N_DEV - 1):
            chunk = lax.rem(my + N_DEV - h, N_DEV)
            rdma = pltpu.make_async_remote_copy(
                src_ref=out_ref.at[chunk],
                dst_ref=out_ref.at[chunk],
                send_sem=send_sems.at[h],
                recv_sem=recv_sems.at[h],
                device_id=(right,),
                device_id_type=pl.DeviceIdType.MESH,
            )
            rdma.start()
            rdma.wait()

    return pl.pallas_call(
        body,
        out_shape=jax.ShapeDtypeStruct((N_DEV, 2, DM, DM), jnp.float32),
        in_specs=[
            pl.BlockSpec(memory_space=pltpu.VMEM),
            pl.BlockSpec(memory_space=pltpu.VMEM),
        ],
        out_specs=pl.BlockSpec(memory_space=pltpu.VMEM),
        scratch_shapes=[
            pltpu.SemaphoreType.DMA((N_DEV - 1,)),
            pltpu.SemaphoreType.DMA((N_DEV - 1,)),
        ],
        compiler_params=pltpu.CompilerParams(collective_id=0),
    )(wq, wo)


def _attention(x, w_full, k_my, v_my):

    def body(x_ref, wq_ref, k_ref, v_ref, wo_ref, o_ref):
        qb = pl.program_id(0)
        h = pl.program_id(1)

        xq = x_ref[0]
        wq = wq_ref[0, 0]
        q = jnp.dot(xq, wq, preferred_element_type=jnp.float32)

        k = k_ref[:, 0, :]
        scores = lax.dot_general(
            q, k, (((1,), (1,)), ((), ())),
            preferred_element_type=jnp.float32,
        ) * SCALE

        qi = qb * QBLK + lax.broadcasted_iota(jnp.int32, (QBLK, SKV), 0)
        ki = lax.broadcasted_iota(jnp.int32, (QBLK, SKV), 1)
        mask = (jnp.abs(qi - ki) <= 128) | (ki < 32) | (qi < 32)
        scores = jnp.where(mask, scores, -1e9)

        m = jnp.max(scores, axis=-1, keepdims=True)
        w = jnp.exp(scores - m)
        w = w / jnp.sum(w, axis=-1, keepdims=True)

        v = v_ref[:, 0, :]
        ctx = jnp.dot(w, v, preferred_element_type=jnp.float32)

        wo = wo_ref[0, 0]
        contrib = jnp.dot(ctx, wo, preferred_element_type=jnp.float32)

        @pl.when(h == 0)
        def _():
            o_ref[0] = contrib

        @pl.when(h != 0)
        def _():
            o_ref[0] += contrib

    grid = (N_QB, HQ_TOTAL)
    return pl.pallas_call(
        body,
        grid=grid,
        out_shape=jax.ShapeDtypeStruct((1, SQ, DM), jnp.float32),
        in_specs=[
            pl.BlockSpec((1, QBLK, DM), lambda qb, h: (0, qb, 0)),
            pl.BlockSpec(
                (1, 1, DM, DH), lambda qb, h: (h // HQ_PER, 0, 0, h % HQ_PER)
            ),
            pl.BlockSpec((SKV, 1, DH), lambda qb, h: (0, h, 0)),
            pl.BlockSpec((SKV, 1, DH), lambda qb, h: (0, h, 0)),
            pl.BlockSpec(
                (1, 1, DH, DM), lambda qb, h: (h // HQ_PER, 1, h % HQ_PER, 0)
            ),
        ],
        out_specs=pl.BlockSpec((1, QBLK, DM), lambda qb, h: (0, qb, 0)),
        compiler_params=pltpu.CompilerParams(
            dimension_semantics=("arbitrary", "arbitrary"),
        ),
    )(x, w_full.at[:, 0], k_my, v_my, w_full.at[:, 1])


def kernel(x, Wq, K_ext, V_ext, Wo):
    w_full = _gather_weights(Wq, Wo)
    my = lax.axis_index("i")
    k_my = lax.dynamic_index_in_dim(K_ext, my, axis=0, keepdims=False)
    v_my = lax.dynamic_index_in_dim(V_ext, my, axis=0, keepdims=False)
    return _attention(x, w_full, k_my, v_my)


# baseline (device time: 994560 ns/iter reference)
import jax
import jax.numpy as jnp
from jax import lax
from jax.experimental import pallas as pl
from jax.experimental.pallas import tpu as pltpu

N_DEV = 4
SQ = 2048
SKV = 2048
DM = 1024
HQ_TOTAL = 32
HQ_PER = 8
DH = 128
SCALE = 0.08838834764831843

QBLK = 256
N_QB = SQ // QBLK


def _gather_weights(wq, wo):

    def body(wq_ref, wo_ref, out_ref, send_sems, recv_sems):
        my = lax.axis_index("i")
        left = lax.rem(my + N_DEV - 1, N_DEV)
        right = lax.rem(my + 1, N_DEV)

        barrier = pltpu.get_barrier_semaphore()
        for nbr in (left, right):
            pl.semaphore_signal(
                barrier, inc=1, device_id=(nbr,),
                device_id_type=pl.DeviceIdType.MESH,
            )
        pl.semaphore_wait(barrier, 2)

        out_ref[pl.ds(my, 1), pl.ds(0, 1)] = wq_ref[...].reshape(1, 1, DM, DM)
        out_ref[pl.ds(my, 1), pl.ds(1, 1)] = wo_ref[...].reshape(1, 1, DM, DM)

        for h in range(N_DEV - 1):
            chunk = lax.rem(my + N_DEV - h, N_DEV)
            rdma = pltpu.make_async_remote_copy(
                src_ref=out_ref.at[chunk],
                dst_ref=out_ref.at[chunk],
                send_sem=send_sems.at[h],
                recv_sem=recv_sems.at[h],
                device_id=(right,),
                device_id_type=pl.DeviceIdType.MESH,
            )
            rdma.start()
            rdma.wait()

    return pl.pallas_call(
        body,
        out_shape=jax.ShapeDtypeStruct((N_DEV, 2, DM, DM), jnp.float32),
        in_specs=[
            pl.BlockSpec(memory_space=pltpu.VMEM),
            pl.BlockSpec(memory_space=pltpu.VMEM),
        ],
        out_specs=pl.BlockSpec(memory_space=pltpu.VMEM),
        scratch_shapes=[
            pltpu.SemaphoreType.DMA((N_DEV - 1,)),
            pltpu.SemaphoreType.DMA((N_DEV - 1,)),
        ],
        compiler_params=pltpu.CompilerParams(collective_id=0),
    )(wq, wo)


def _attention(x, w_full, k_my, v_my):

    def body(x_ref, wq_ref, k_ref, v_ref, wo_ref, o_ref):
        qb = pl.program_id(0)
        h = pl.program_id(1)

        xq = x_ref[0]
        wq = wq_ref[0, 0]
        q = jnp.dot(xq, wq, preferred_element_type=jnp.float32)

        k = k_ref[...]
        scores = lax.dot_general(
            q, k, (((1,), (1,)), ((), ())),
            preferred_element_type=jnp.float32,
        ) * SCALE

        qi = qb * QBLK + lax.broadcasted_iota(jnp.int32, (QBLK, SKV), 0)
        ki = lax.broadcasted_iota(jnp.int32, (QBLK, SKV), 1)
        mask = (jnp.abs(qi - ki) <= 128) | (ki < 32) | (qi < 32)
        scores = jnp.where(mask, scores, -1e9)

        m = jnp.max(scores, axis=-1, keepdims=True)
        w = jnp.exp(scores - m)
        w = w / jnp.sum(w, axis=-1, keepdims=True)

        v = v_ref[...]
        ctx = jnp.dot(w, v, preferred_element_type=jnp.float32)

        wo = wo_ref[0, 0]
        contrib = jnp.dot(ctx, wo, preferred_element_type=jnp.float32)

        @pl.when(h == 0)
        def _():
            o_ref[0] = contrib

        @pl.when(h != 0)
        def _():
            o_ref[0] += contrib

    grid = (N_QB, HQ_TOTAL)
    return pl.pallas_call(
        body,
        grid=grid,
        out_shape=jax.ShapeDtypeStruct((1, SQ, DM), jnp.float32),
        in_specs=[
            pl.BlockSpec((1, QBLK, DM), lambda qb, h: (0, qb, 0)),
            pl.BlockSpec(
                (1, 1, DM, DH), lambda qb, h: (h // HQ_PER, 0, 0, h % HQ_PER)
            ),
            pl.BlockSpec((SKV, DH), lambda qb, h: (0, h)),
            pl.BlockSpec((SKV, DH), lambda qb, h: (0, h)),
            pl.BlockSpec(
                (1, 1, DH, DM), lambda qb, h: (h // HQ_PER, 1, h % HQ_PER, 0)
            ),
        ],
        out_specs=pl.BlockSpec((1, QBLK, DM), lambda qb, h: (0, qb, 0)),
        compiler_params=pltpu.CompilerParams(
            dimension_semantics=("arbitrary", "arbitrary"),
        ),
    )(x, w_full, k_my, v_my, w_full)


def kernel(x, Wq, K_ext, V_ext, Wo):
    w_full = _gather_weights(Wq, Wo)
    my = lax.axis_index("i")
    k_my = lax.dynamic_index_in_dim(K_ext, my, axis=0, keepdims=False)
    v_my = lax.dynamic_index_in_dim(V_ext, my, axis=0, keepdims=False)
    k_my = k_my.reshape(SKV, HQ_TOTAL * DH)
    v_my = v_my.reshape(SKV, HQ_TOTAL * DH)
    return _attention(x, w_full, k_my, v_my)
